# Initial kernel scaffold; baseline (speedup 1.0000x reference)
import jax
import jax.numpy as jnp
from jax.experimental import pallas as pl


def kernel(hidden_states, gate, gate_up_proj, down_proj):
    raise NotImplementedError("write your pallas kernel here")


# speedup vs baseline: 7.6196x; 7.6196x over previous
import functools

import jax
import jax.numpy as jnp
from jax import lax
from jax.experimental import pallas as pl
from jax.experimental.pallas import tpu as pltpu

E = 64
H = 1024
I = 768
NT = 2048
TB = 256
TM = 128
G = NT // TM + E


def _router_body(x_ref, gate_ref, lg_ref, eid_ref, w_ref):
    l = lax.dot_general(x_ref[:, :], gate_ref[:, :],
                        (((1,), (1,)), ((), ())),
                        preferred_element_type=jnp.float32)
    lg_ref[:, :] = l
    m = jnp.max(l, axis=1, keepdims=True)
    s = jnp.sum(jnp.exp(l - m), axis=1)
    w_ref[0, 0, :] = 1.0 / s
    eid_ref[0, 0, :] = jnp.argmax(l, axis=1).astype(jnp.int32)


def _gemm_body(e_ref, base_ref, cnt_ref, gidx_ref, wt_ref,
               x_ref, gup_ref, dn_ref, out_ref, xa_ref, ya_ref):
    g = pl.program_id(0)
    base = base_ref[g]
    cnt = cnt_ref[g]

    def gather(i, _):
        src = gidx_ref[jnp.minimum(base + i, NT - 1)]
        xa_ref[pl.ds(i, 1), :] = x_ref[pl.ds(src, 1), :]
        return 0

    lax.fori_loop(0, cnt, gather, 0)

    @pl.when(cnt > 0)
    def _():
        a = xa_ref[:, :]
        h = lax.dot_general(a, gup_ref[0], (((1,), (1,)), ((), ())),
                            preferred_element_type=jnp.float32)
        hg = h[:, :I]
        hu = h[:, I:]
        inter = hg * jax.nn.sigmoid(hg) * hu
        ya_ref[:, :] = lax.dot_general(inter, dn_ref[0],
                                       (((1,), (1,)), ((), ())),
                                       preferred_element_type=jnp.float32)

        def scatter(i, _):
            dst = gidx_ref[jnp.minimum(base + i, NT - 1)]
            w = wt_ref[dst]
            out_ref[pl.ds(dst, 1), :] = ya_ref[pl.ds(i, 1), :] * w
            return 0

        lax.fori_loop(0, cnt, scatter, 0)


def kernel(hidden_states, gate, gate_up_proj, down_proj):
    bsz, seq, hd = hidden_states.shape
    x = hidden_states.reshape(NT, H)

    logits, eid3, wt3 = pl.pallas_call(
        _router_body,
        grid=(NT // TB,),
        in_specs=[
            pl.BlockSpec((TB, H), lambda t: (t, 0)),
            pl.BlockSpec((E, H), lambda t: (0, 0)),
        ],
        out_specs=[
            pl.BlockSpec((TB, E), lambda t: (t, 0)),
            pl.BlockSpec((1, 1, TB), lambda t: (t, 0, 0)),
            pl.BlockSpec((1, 1, TB), lambda t: (t, 0, 0)),
        ],
        out_shape=[
            jax.ShapeDtypeStruct((NT, E), jnp.float32),
            jax.ShapeDtypeStruct((NT // TB, 1, TB), jnp.int32),
            jax.ShapeDtypeStruct((NT // TB, 1, TB), jnp.float32),
        ],
    )(x, gate)
    e_flat = eid3.reshape(NT)
    w_tok = wt3.reshape(NT)

    counts = jnp.zeros((E,), jnp.int32).at[e_flat].add(1)
    offsets = jnp.concatenate([jnp.zeros((1,), jnp.int32),
                               jnp.cumsum(counts)[:-1].astype(jnp.int32)])
    gather_idx = jnp.argsort(e_flat).astype(jnp.int32)
    tiles_per = (counts + TM - 1) // TM
    ic = jnp.cumsum(tiles_per).astype(jnp.int32)
    total = ic[-1]
    g_idx = jnp.arange(G, dtype=jnp.int32)
    e_of_g = jnp.minimum(
        jnp.searchsorted(ic, g_idx, side='right'), E - 1).astype(jnp.int32)
    e_last = jnp.max(e_flat).astype(jnp.int32)
    e_of_g = jnp.where(g_idx < total, e_of_g, e_last)
    tile_starts = ic[e_of_g] - tiles_per[e_of_g]
    j = g_idx - tile_starts
    base = jnp.clip(offsets[e_of_g] + j * TM, 0, NT - 1).astype(jnp.int32)
    cnt = jnp.clip(counts[e_of_g] - j * TM, 0, TM).astype(jnp.int32)

    grid_spec = pltpu.PrefetchScalarGridSpec(
        num_scalar_prefetch=5,
        grid=(G,),
        in_specs=[
            pl.BlockSpec((NT, H), lambda g, e, b, c, gi, w: (0, 0)),
            pl.BlockSpec((1, 2 * I, H), lambda g, e, b, c, gi, w: (e[g], 0, 0)),
            pl.BlockSpec((1, H, I), lambda g, e, b, c, gi, w: (e[g], 0, 0)),
        ],
        out_specs=pl.BlockSpec((NT, H), lambda g, e, b, c, gi, w: (0, 0)),
        scratch_shapes=[pltpu.VMEM((TM, H), jnp.float32),
                        pltpu.VMEM((TM, H), jnp.float32)],
    )
    out = pl.pallas_call(
        _gemm_body,
        grid_spec=grid_spec,
        out_shape=jax.ShapeDtypeStruct((NT, H), jnp.float32),
        compiler_params=pltpu.CompilerParams(
            dimension_semantics=("arbitrary",)),
    )(e_of_g, base, cnt, gather_idx, w_tok, x, gate_up_proj, down_proj)

    return out.reshape(bsz, seq, hd), logits


# speedup vs baseline: 8.7270x; 1.1453x over previous
import jax
import jax.numpy as jnp
from jax import lax
from jax.experimental import pallas as pl
from jax.experimental.pallas import tpu as pltpu

E = 64
H = 1024
I = 768
NT = 2048
RT = 256
TM = 128


def _body(x_ref, gate_ref, gup_ref, dn_ref, out_ref, lg_ref,
          xa_ref, ya_ref, lrank_ref, posv_ref, wtv_ref, cntv_ref, offv_ref,
          gidx_s, pos_s, wt_s, cnt_s, off_s, sem):
    e = pl.program_id(0)

    @pl.when(e == 0)
    def _prologue():
        l = lax.dot_general(x_ref[:, :], gate_ref[:, :],
                            (((1,), (1,)), ((), ())),
                            preferred_element_type=jnp.float32)
        lg_ref[:, :] = l
        m = jnp.max(l, axis=1, keepdims=True)
        s = jnp.sum(jnp.exp(l - m), axis=1)
        w = 1.0 / s
        eid = jnp.argmax(l, axis=1).astype(jnp.int32)

        iota_e = lax.broadcasted_iota(jnp.int32, (RT, E), 1)
        tril = (lax.broadcasted_iota(jnp.int32, (RT, RT), 0) >
                lax.broadcasted_iota(jnp.int32, (RT, RT), 1)).astype(jnp.float32)
        carry = jnp.zeros((1, E), jnp.float32)
        for t in range(NT // RT):
            eid_t = eid[t * RT:(t + 1) * RT]
            oh = (eid_t[:, None] == iota_e).astype(jnp.float32)
            ranks = lax.dot_general(tril, oh, (((1,), (0,)), ((), ())),
                                    preferred_element_type=jnp.float32)
            lrank_ref[0, t * RT:(t + 1) * RT] = (
                jnp.sum(oh * ranks, axis=1) + jnp.sum(oh * carry, axis=1))
            carry = carry + jnp.sum(oh, axis=0, keepdims=True)
        triu = (lax.broadcasted_iota(jnp.int32, (E, E), 0) <
                lax.broadcasted_iota(jnp.int32, (E, E), 1)).astype(jnp.float32)
        offs = lax.dot_general(carry, triu, (((1,), (0,)), ((), ())),
                               preferred_element_type=jnp.float32)
        cntv_ref[0, :] = carry[0].astype(jnp.int32)
        offv_ref[0, :] = offs[0].astype(jnp.int32)
        oh_full = (eid[:, None] ==
                   lax.broadcasted_iota(jnp.int32, (NT, E), 1)).astype(jnp.float32)
        off_tok = jnp.sum(oh_full * offs, axis=1)
        posv_ref[0, :] = (lrank_ref[0, :] + off_tok).astype(jnp.int32)
        wtv_ref[0, :] = w

        for src, dst in ((posv_ref, pos_s), (wtv_ref, wt_s),
                         (cntv_ref, cnt_s), (offv_ref, off_s)):
            cp = pltpu.make_async_copy(src, dst, sem)
            cp.start()
            cp.wait()

        def inv(t, _):
            gidx_s[0, pos_s[0, t]] = t
            return 0

        lax.fori_loop(0, NT, inv, 0)

    start = off_s[0, e]
    cnt_e = cnt_s[0, e]
    n_tiles = (cnt_e + TM - 1) // TM

    def tile_body(j, _):
        base = start + j * TM
        rows = jnp.minimum(cnt_e - j * TM, TM)

        def gather(r, _):
            src = gidx_s[0, base + r]
            xa_ref[pl.ds(r, 1), :] = x_ref[pl.ds(src, 1), :]
            return 0

        lax.fori_loop(0, rows, gather, 0)
        h = lax.dot_general(xa_ref[:, :], gup_ref[0], (((1,), (1,)), ((), ())),
                            preferred_element_type=jnp.float32)
        hg = h[:, :I]
        hu = h[:, I:]
        inter = hg * jax.nn.sigmoid(hg) * hu
        ya_ref[:, :] = lax.dot_general(inter, dn_ref[0],
                                       (((1,), (1,)), ((), ())),
                                       preferred_element_type=jnp.float32)

        def scatter(r, _):
            dst = gidx_s[0, base + r]
            out_ref[pl.ds(dst, 1), :] = ya_ref[pl.ds(r, 1), :] * wt_s[0, dst]
            return 0

        lax.fori_loop(0, rows, scatter, 0)
        return 0

    lax.fori_loop(0, n_tiles, tile_body, 0)


def kernel(hidden_states, gate, gate_up_proj, down_proj):
    bsz, seq, hd = hidden_states.shape
    x = hidden_states.reshape(NT, H)

    out, logits = pl.pallas_call(
        _body,
        grid=(E,),
        in_specs=[
            pl.BlockSpec((NT, H), lambda e: (0, 0)),
            pl.BlockSpec((E, H), lambda e: (0, 0)),
            pl.BlockSpec((1, 2 * I, H), lambda e: (e, 0, 0)),
            pl.BlockSpec((1, H, I), lambda e: (e, 0, 0)),
        ],
        out_specs=[
            pl.BlockSpec((NT, H), lambda e: (0, 0)),
            pl.BlockSpec((NT, E), lambda e: (0, 0)),
        ],
        out_shape=[
            jax.ShapeDtypeStruct((NT, H), jnp.float32),
            jax.ShapeDtypeStruct((NT, E), jnp.float32),
        ],
        scratch_shapes=[
            pltpu.VMEM((TM, H), jnp.float32),
            pltpu.VMEM((TM, H), jnp.float32),
            pltpu.VMEM((1, NT), jnp.float32),
            pltpu.VMEM((1, NT), jnp.int32),
            pltpu.VMEM((1, NT), jnp.float32),
            pltpu.VMEM((1, E), jnp.int32),
            pltpu.VMEM((1, E), jnp.int32),
            pltpu.SMEM((1, NT), jnp.int32),
            pltpu.SMEM((1, NT), jnp.int32),
            pltpu.SMEM((1, NT), jnp.float32),
            pltpu.SMEM((1, E), jnp.int32),
            pltpu.SMEM((1, E), jnp.int32),
            pltpu.SemaphoreType.DMA,
        ],
        compiler_params=pltpu.CompilerParams(
            dimension_semantics=("arbitrary",)),
    )(x, gate, gate_up_proj, down_proj)

    return out.reshape(bsz, seq, hd), logits


# speedup vs baseline: 9.8315x; 1.1266x over previous
import jax
import jax.numpy as jnp
from jax import lax
from jax.experimental import pallas as pl
from jax.experimental.pallas import tpu as pltpu

E = 64
H = 1024
I = 768
NT = 2048
RT = 256
TM = 128
NB = 4


def _body(x_ref, gate_ref, gup_hbm, dn_hbm, out_ref, lg_ref,
          xa_ref, ya_ref, lrank_ref, posv_ref, wtv_ref, cntv_ref, offv_ref,
          gidx_s, pos_s, wt_s, cnt_s, off_s, sem, gup_buf, dn_buf, gsem, dsem):
    e = pl.program_id(0)

    def gup_copy(src_e, slot):
        return pltpu.make_async_copy(gup_hbm.at[src_e], gup_buf.at[slot],
                                     gsem.at[slot])

    def dn_copy(src_e, slot):
        return pltpu.make_async_copy(dn_hbm.at[src_e], dn_buf.at[slot],
                                     dsem.at[slot])

    @pl.when(e == 0)
    def _prefetch():
        for k in range(NB):
            gup_copy(k, k).start()
            dn_copy(k, k).start()

    @pl.when(e == 0)
    def _prologue():
        l = lax.dot_general(x_ref[:, :], gate_ref[:, :],
                            (((1,), (1,)), ((), ())),
                            preferred_element_type=jnp.float32)
        lg_ref[:, :] = l
        m = jnp.max(l, axis=1, keepdims=True)
        s = jnp.sum(jnp.exp(l - m), axis=1)
        w = 1.0 / s
        eid = jnp.argmax(l, axis=1).astype(jnp.int32)

        iota_e = lax.broadcasted_iota(jnp.int32, (RT, E), 1)
        tril = (lax.broadcasted_iota(jnp.int32, (RT, RT), 0) >
                lax.broadcasted_iota(jnp.int32, (RT, RT), 1)).astype(jnp.float32)
        carry = jnp.zeros((1, E), jnp.float32)
        for t in range(NT // RT):
            eid_t = eid[t * RT:(t + 1) * RT]
            oh = (eid_t[:, None] == iota_e).astype(jnp.float32)
            ranks = lax.dot_general(tril, oh, (((1,), (0,)), ((), ())),
                                    preferred_element_type=jnp.float32)
            lrank_ref[0, t * RT:(t + 1) * RT] = (
                jnp.sum(oh * ranks, axis=1) + jnp.sum(oh * carry, axis=1))
            carry = carry + jnp.sum(oh, axis=0, keepdims=True)
        triu = (lax.broadcasted_iota(jnp.int32, (E, E), 0) <
                lax.broadcasted_iota(jnp.int32, (E, E), 1)).astype(jnp.float32)
        offs = lax.dot_general(carry, triu, (((1,), (0,)), ((), ())),
                               preferred_element_type=jnp.float32)
        cntv_ref[0, :] = carry[0].astype(jnp.int32)
        offv_ref[0, :] = offs[0].astype(jnp.int32)
        oh_full = (eid[:, None] ==
                   lax.broadcasted_iota(jnp.int32, (NT, E), 1)).astype(jnp.float32)
        off_tok = jnp.sum(oh_full * offs, axis=1)
        posv_ref[0, :] = (lrank_ref[0, :] + off_tok).astype(jnp.int32)
        wtv_ref[0, :] = w

        for src, dst in ((posv_ref, pos_s), (wtv_ref, wt_s),
                         (cntv_ref, cnt_s), (offv_ref, off_s)):
            cp = pltpu.make_async_copy(src, dst, sem)
            cp.start()
            cp.wait()

        def inv(t, _):
            gidx_s[0, pos_s[0, t]] = t
            return 0

        lax.fori_loop(0, NT, inv, 0)

    slot = lax.rem(e, NB)
    gup_copy(e, slot).wait()
    dn_copy(e, slot).wait()

    start = off_s[0, e]
    cnt_e = cnt_s[0, e]
    n_tiles = (cnt_e + TM - 1) // TM

    def tile_body(j, _):
        base = start + j * TM
        rows = jnp.minimum(cnt_e - j * TM, TM)

        def gather(r, _):
            src = gidx_s[0, base + r]
            xa_ref[pl.ds(r, 1), :] = x_ref[pl.ds(src, 1), :]
            return 0

        lax.fori_loop(0, rows, gather, 0)
        h = lax.dot_general(xa_ref[:, :], gup_buf[slot],
                            (((1,), (1,)), ((), ())),
                            preferred_element_type=jnp.float32)
        hg = h[:, :I]
        hu = h[:, I:]
        inter = hg * jax.nn.sigmoid(hg) * hu
        ya_ref[:, :] = lax.dot_general(inter, dn_buf[slot],
                                       (((1,), (1,)), ((), ())),
                                       preferred_element_type=jnp.float32)

        def scatter(r, _):
            dst = gidx_s[0, base + r]
            out_ref[pl.ds(dst, 1), :] = ya_ref[pl.ds(r, 1), :] * wt_s[0, dst]
            return 0

        lax.fori_loop(0, rows, scatter, 0)
        return 0

    lax.fori_loop(0, n_tiles, tile_body, 0)

    @pl.when(e + NB < E)
    def _refill():
        gup_copy(e + NB, slot).start()
        dn_copy(e + NB, slot).start()


def kernel(hidden_states, gate, gate_up_proj, down_proj):
    bsz, seq, hd = hidden_states.shape
    x = hidden_states.reshape(NT, H)

    out, logits = pl.pallas_call(
        _body,
        grid=(E,),
        in_specs=[
            pl.BlockSpec((NT, H), lambda e: (0, 0)),
            pl.BlockSpec((E, H), lambda e: (0, 0)),
            pl.BlockSpec(memory_space=pl.ANY),
            pl.BlockSpec(memory_space=pl.ANY),
        ],
        out_specs=[
            pl.BlockSpec((NT, H), lambda e: (0, 0)),
            pl.BlockSpec((NT, E), lambda e: (0, 0)),
        ],
        out_shape=[
            jax.ShapeDtypeStruct((NT, H), jnp.float32),
            jax.ShapeDtypeStruct((NT, E), jnp.float32),
        ],
        scratch_shapes=[
            pltpu.VMEM((TM, H), jnp.float32),
            pltpu.VMEM((TM, H), jnp.float32),
            pltpu.VMEM((1, NT), jnp.float32),
            pltpu.VMEM((1, NT), jnp.int32),
            pltpu.VMEM((1, NT), jnp.float32),
            pltpu.VMEM((1, E), jnp.int32),
            pltpu.VMEM((1, E), jnp.int32),
            pltpu.SMEM((1, NT), jnp.int32),
            pltpu.SMEM((1, NT), jnp.int32),
            pltpu.SMEM((1, NT), jnp.float32),
            pltpu.SMEM((1, E), jnp.int32),
            pltpu.SMEM((1, E), jnp.int32),
            pltpu.SemaphoreType.DMA,
            pltpu.VMEM((NB, 2 * I, H), jnp.float32),
            pltpu.VMEM((NB, H, I), jnp.float32),
            pltpu.SemaphoreType.DMA((NB,)),
            pltpu.SemaphoreType.DMA((NB,)),
        ],
        compiler_params=pltpu.CompilerParams(
            dimension_semantics=("arbitrary",)),
    )(x, gate, gate_up_proj, down_proj)

    return out.reshape(bsz, seq, hd), logits
